# baseline (speedup 1.0000x reference)
import functools

import jax
import jax.numpy as jnp
from jax import lax
from jax.experimental import pallas as pl
from jax.experimental.pallas import tpu as pltpu
from jax.experimental.pallas import tpu_sc as plsc

N = 10000
E = 320000
D = 128
DOUT = 64
G = 64
NP = 10240
BLK = 512
NBLK = NP // BLK
CHUNK = 128

NC = 2
NS = 16
NW = NC * NS
EPW_CHUNKS = 80
EPW = EPW_CHUNKS * CHUNK
EPAD = EPW * NW
RPS = NP // NS


def _edge_agg(table, src, dst, zeros):
  mesh = plsc.VectorSubcoreMesh(core_axis_name="c", subcore_axis_name="s")

  @functools.partial(
      pl.kernel,
      mesh=mesh,
      out_type=jax.ShapeDtypeStruct((NC, NP, D), jnp.float32),
      scratch_types=[
          pltpu.VMEM((CHUNK,), jnp.int32),
          pltpu.VMEM((CHUNK,), jnp.int32),
          pltpu.VMEM((CHUNK, D), jnp.float32),
          pltpu.VMEM_SHARED((NP, D), jnp.float32),
          pltpu.SemaphoreType.DMA,
      ],
  )
  def agg(table_h, src_h, dst_h, zeros_h, out_h, src_v, dst_v, rows_v, acc,
          sem):
    c = lax.axis_index("c")
    s = lax.axis_index("s")
    wid = s * NC + c
    zbase = s * RPS
    pltpu.sync_copy(zeros_h.at[pl.ds(zbase, RPS)], acc.at[pl.ds(zbase, RPS)])
    plsc.subcore_barrier()

    def body(k, carry):
      off = (wid * EPW_CHUNKS + k) * CHUNK
      pltpu.sync_copy(src_h.at[pl.ds(off, CHUNK)], src_v)
      pltpu.sync_copy(dst_h.at[pl.ds(off, CHUNK)], dst_v)
      pltpu.async_copy(table_h.at[src_v], rows_v, sem).wait()
      pltpu.sync_copy(rows_v, acc.at[dst_v], add=True)
      return carry

    lax.fori_loop(0, EPW_CHUNKS, body, 0)
    plsc.subcore_barrier()
    pltpu.sync_copy(acc.at[pl.ds(zbase, RPS)], out_h.at[c, pl.ds(zbase, RPS)])

  return agg(table, src, dst, zeros)


def _mm2(x, Wn, Wr, b):
  def body(x_ref, wn_ref, wr_ref, b_ref, xn_ref, xr_ref):
    xb = x_ref[...]
    xn_ref[...] = jnp.dot(xb, wn_ref[...], preferred_element_type=jnp.float32)
    xr_ref[...] = (
        jnp.dot(xb, wr_ref[...], preferred_element_type=jnp.float32)
        + b_ref[...])

  return pl.pallas_call(
      body,
      grid=(NBLK,),
      in_specs=[
          pl.BlockSpec((BLK, D), lambda i: (i, 0)),
          pl.BlockSpec((D, D), lambda i: (0, 0)),
          pl.BlockSpec((D, D), lambda i: (0, 0)),
          pl.BlockSpec((1, D), lambda i: (0, 0)),
      ],
      out_specs=[
          pl.BlockSpec((BLK, D), lambda i: (i, 0)),
          pl.BlockSpec((BLK, D), lambda i: (i, 0)),
      ],
      out_shape=[
          jax.ShapeDtypeStruct((NP, D), jnp.float32),
          jax.ShapeDtypeStruct((NP, D), jnp.float32),
      ],
  )(x, Wn, Wr, b.reshape(1, D))


def _combine_mm2(p0, p1, xr, Wn, Wr, b):
  def body(p0_ref, p1_ref, xr_ref, wn_ref, wr_ref, b_ref, xn_ref, xr2_ref):
    i = pl.program_id(0)
    h = jnp.maximum(p0_ref[...] + p1_ref[...] + xr_ref[...], 0.0)
    rows = i * BLK + lax.broadcasted_iota(jnp.int32, (BLK, D), 0)
    h = jnp.where(rows < N, h, 0.0)
    xn_ref[...] = jnp.dot(h, wn_ref[...], preferred_element_type=jnp.float32)
    xr2_ref[...] = (
        jnp.dot(h, wr_ref[...], preferred_element_type=jnp.float32)
        + b_ref[...])

  return pl.pallas_call(
      body,
      grid=(NBLK,),
      in_specs=[
          pl.BlockSpec((BLK, D), lambda i: (i, 0)),
          pl.BlockSpec((BLK, D), lambda i: (i, 0)),
          pl.BlockSpec((BLK, D), lambda i: (i, 0)),
          pl.BlockSpec((D, D), lambda i: (0, 0)),
          pl.BlockSpec((D, D), lambda i: (0, 0)),
          pl.BlockSpec((1, D), lambda i: (0, 0)),
      ],
      out_specs=[
          pl.BlockSpec((BLK, D), lambda i: (i, 0)),
          pl.BlockSpec((BLK, D), lambda i: (i, 0)),
      ],
      out_shape=[
          jax.ShapeDtypeStruct((NP, D), jnp.float32),
          jax.ShapeDtypeStruct((NP, D), jnp.float32),
      ],
  )(p0, p1, xr, Wn, Wr, b.reshape(1, D))


def _pool_head(p0, p1, xr, batch3d, W3, b3):
  def body(p0_ref, p1_ref, xr_ref, bat_ref, w3_ref, b3_ref, out_ref, acc_ref):
    i = pl.program_id(0)

    @pl.when(i == 0)
    def _():
      acc_ref[...] = jnp.zeros_like(acc_ref)

    h = jnp.maximum(p0_ref[...] + p1_ref[...] + xr_ref[...], 0.0)
    ids = bat_ref[...].reshape(1, BLK)
    onehot = (lax.broadcasted_iota(jnp.int32, (G, BLK), 0)
              == jnp.broadcast_to(ids, (G, BLK))).astype(jnp.float32)
    acc_ref[...] += jnp.dot(onehot, h, preferred_element_type=jnp.float32)

    @pl.when(i == NBLK - 1)
    def _():
      out_ref[...] = (
          jnp.dot(acc_ref[...], w3_ref[...],
                  preferred_element_type=jnp.float32) + b3_ref[...])

  return pl.pallas_call(
      body,
      grid=(NBLK,),
      in_specs=[
          pl.BlockSpec((BLK, D), lambda i: (i, 0)),
          pl.BlockSpec((BLK, D), lambda i: (i, 0)),
          pl.BlockSpec((BLK, D), lambda i: (i, 0)),
          pl.BlockSpec((1, 1, BLK), lambda i: (i, 0, 0)),
          pl.BlockSpec((D, DOUT), lambda i: (0, 0)),
          pl.BlockSpec((1, DOUT), lambda i: (0, 0)),
      ],
      out_specs=pl.BlockSpec((G, DOUT), lambda i: (0, 0)),
      out_shape=jax.ShapeDtypeStruct((G, DOUT), jnp.float32),
      scratch_shapes=[pltpu.VMEM((G, D), jnp.float32)],
  )(p0, p1, xr, batch3d, W3, b3.reshape(1, DOUT))


def kernel(x, edge_index, batch, Wn1, Wr1, b1, Wn2, Wr2, b2, W3, b3):
  x_pad = jnp.pad(x, ((0, NP - N), (0, 0)))
  fill = jnp.full((EPAD - E,), NP - 1, jnp.int32)
  src_p = jnp.concatenate([edge_index[0], fill])
  dst_p = jnp.concatenate([edge_index[1], fill])
  batch3d = jnp.pad(batch, (0, NP - N),
                    constant_values=G).reshape(NBLK, 1, BLK)
  zeros = jnp.zeros((NP, D), jnp.float32)

  xn1, xr1 = _mm2(x_pad, Wn1, Wr1, b1)
  p = _edge_agg(xn1, src_p, dst_p, zeros)
  xn2, xr2 = _combine_mm2(p[0], p[1], xr1, Wn2, Wr2, b2)
  p2 = _edge_agg(xn2, src_p, dst_p, zeros)
  return _pool_head(p2[0], p2[1], xr2, batch3d, W3, b3)


# speedup vs baseline: 1.4799x; 1.4799x over previous
import functools

import jax
import jax.numpy as jnp
from jax import lax
from jax.experimental import pallas as pl
from jax.experimental.pallas import tpu as pltpu
from jax.experimental.pallas import tpu_sc as plsc

N = 10000
E = 320000
D = 128
DOUT = 64
G = 64
NP = 10240
BLK = 512
NBLK = NP // BLK
CHUNK = 128

NC = 2
NS = 16
NW = NC * NS
EPW_CHUNKS = -(-E // (NW * CHUNK))
EPW = EPW_CHUNKS * CHUNK
EPAD = EPW * NW
RPS = NP // NS


def _edge_agg(table, src, dst, zeros):
  mesh = plsc.VectorSubcoreMesh(core_axis_name="c", subcore_axis_name="s")

  @functools.partial(
      pl.kernel,
      mesh=mesh,
      out_type=jax.ShapeDtypeStruct((NC, NP, D), jnp.float32),
      scratch_types=[
          pltpu.VMEM((CHUNK,), jnp.int32),
          pltpu.VMEM((CHUNK,), jnp.int32),
          pltpu.VMEM((CHUNK, D), jnp.float32),
          pltpu.VMEM_SHARED((NP, D), jnp.float32),
          pltpu.SemaphoreType.DMA,
      ],
  )
  def agg(table_h, src_h, dst_h, zeros_h, out_h, src_v, dst_v, rows_v, acc,
          sem):
    c = lax.axis_index("c")
    s = lax.axis_index("s")
    wid = s * NC + c
    zbase = s * RPS
    pltpu.sync_copy(zeros_h.at[pl.ds(zbase, RPS)], acc.at[pl.ds(zbase, RPS)])
    plsc.subcore_barrier()

    def body(k, carry):
      pltpu.sync_copy(src_h.at[wid, k], src_v)
      pltpu.sync_copy(dst_h.at[wid, k], dst_v)
      pltpu.async_copy(table_h.at[src_v], rows_v, sem).wait()
      pltpu.sync_copy(rows_v, acc.at[dst_v], add=True)
      return carry

    lax.fori_loop(0, EPW_CHUNKS, body, 0)
    plsc.subcore_barrier()
    pltpu.sync_copy(acc.at[pl.ds(zbase, RPS)], out_h.at[c, pl.ds(zbase, RPS)])

  return agg(table, src, dst, zeros)


def _mm2(x, Wn, Wr, b):
  def body(x_ref, wn_ref, wr_ref, b_ref, xn_ref, xr_ref):
    xb = x_ref[...]
    xn_ref[...] = jnp.dot(xb, wn_ref[...], preferred_element_type=jnp.float32)
    xr_ref[...] = (
        jnp.dot(xb, wr_ref[...], preferred_element_type=jnp.float32)
        + b_ref[...])

  return pl.pallas_call(
      body,
      grid=(NBLK,),
      in_specs=[
          pl.BlockSpec((BLK, D), lambda i: (i, 0)),
          pl.BlockSpec((D, D), lambda i: (0, 0)),
          pl.BlockSpec((D, D), lambda i: (0, 0)),
          pl.BlockSpec((1, D), lambda i: (0, 0)),
      ],
      out_specs=[
          pl.BlockSpec((BLK, D), lambda i: (i, 0)),
          pl.BlockSpec((BLK, D), lambda i: (i, 0)),
      ],
      out_shape=[
          jax.ShapeDtypeStruct((NP, D), jnp.float32),
          jax.ShapeDtypeStruct((NP, D), jnp.float32),
      ],
  )(x, Wn, Wr, b.reshape(1, D))


def _combine_mm2(p0, p1, xr, Wn, Wr, b):
  def body(p0_ref, p1_ref, xr_ref, wn_ref, wr_ref, b_ref, xn_ref, xr2_ref):
    i = pl.program_id(0)
    h = jnp.maximum(p0_ref[...] + p1_ref[...] + xr_ref[...], 0.0)
    rows = i * BLK + lax.broadcasted_iota(jnp.int32, (BLK, D), 0)
    h = jnp.where(rows < N, h, 0.0)
    xn_ref[...] = jnp.dot(h, wn_ref[...], preferred_element_type=jnp.float32)
    xr2_ref[...] = (
        jnp.dot(h, wr_ref[...], preferred_element_type=jnp.float32)
        + b_ref[...])

  return pl.pallas_call(
      body,
      grid=(NBLK,),
      in_specs=[
          pl.BlockSpec((BLK, D), lambda i: (i, 0)),
          pl.BlockSpec((BLK, D), lambda i: (i, 0)),
          pl.BlockSpec((BLK, D), lambda i: (i, 0)),
          pl.BlockSpec((D, D), lambda i: (0, 0)),
          pl.BlockSpec((D, D), lambda i: (0, 0)),
          pl.BlockSpec((1, D), lambda i: (0, 0)),
      ],
      out_specs=[
          pl.BlockSpec((BLK, D), lambda i: (i, 0)),
          pl.BlockSpec((BLK, D), lambda i: (i, 0)),
      ],
      out_shape=[
          jax.ShapeDtypeStruct((NP, D), jnp.float32),
          jax.ShapeDtypeStruct((NP, D), jnp.float32),
      ],
  )(p0, p1, xr, Wn, Wr, b.reshape(1, D))


def _pool_head(p0, p1, xr, batch3d, W3, b3):
  def body(p0_ref, p1_ref, xr_ref, bat_ref, w3_ref, b3_ref, out_ref, acc_ref):
    i = pl.program_id(0)

    @pl.when(i == 0)
    def _():
      acc_ref[...] = jnp.zeros_like(acc_ref)

    h = jnp.maximum(p0_ref[...] + p1_ref[...] + xr_ref[...], 0.0)
    ids = bat_ref[...].reshape(1, BLK)
    onehot = (lax.broadcasted_iota(jnp.int32, (G, BLK), 0)
              == jnp.broadcast_to(ids, (G, BLK))).astype(jnp.float32)
    acc_ref[...] += jnp.dot(onehot, h, preferred_element_type=jnp.float32)

    @pl.when(i == NBLK - 1)
    def _():
      out_ref[...] = (
          jnp.dot(acc_ref[...], w3_ref[...],
                  preferred_element_type=jnp.float32) + b3_ref[...])

  return pl.pallas_call(
      body,
      grid=(NBLK,),
      in_specs=[
          pl.BlockSpec((BLK, D), lambda i: (i, 0)),
          pl.BlockSpec((BLK, D), lambda i: (i, 0)),
          pl.BlockSpec((BLK, D), lambda i: (i, 0)),
          pl.BlockSpec((1, 1, BLK), lambda i: (i, 0, 0)),
          pl.BlockSpec((D, DOUT), lambda i: (0, 0)),
          pl.BlockSpec((1, DOUT), lambda i: (0, 0)),
      ],
      out_specs=pl.BlockSpec((G, DOUT), lambda i: (0, 0)),
      out_shape=jax.ShapeDtypeStruct((G, DOUT), jnp.float32),
      scratch_shapes=[pltpu.VMEM((G, D), jnp.float32)],
  )(p0, p1, xr, batch3d, W3, b3.reshape(1, DOUT))


def kernel(x, edge_index, batch, Wn1, Wr1, b1, Wn2, Wr2, b2, W3, b3):
  x_pad = jnp.pad(x, ((0, NP - N), (0, 0)))
  fill = jnp.full((EPAD - E,), NP - 1, jnp.int32)
  src_p = jnp.concatenate([edge_index[0], fill]).reshape(NW, EPW_CHUNKS, CHUNK)
  dst_p = jnp.concatenate([edge_index[1], fill]).reshape(NW, EPW_CHUNKS, CHUNK)
  batch3d = jnp.pad(batch, (0, NP - N),
                    constant_values=G).reshape(NBLK, 1, BLK)
  zeros = jnp.zeros((NP, D), jnp.float32)

  xn1, xr1 = _mm2(x_pad, Wn1, Wr1, b1)
  p = _edge_agg(xn1, src_p, dst_p, zeros)
  xn2, xr2 = _combine_mm2(p[0], p[1], xr1, Wn2, Wr2, b2)
  p2 = _edge_agg(xn2, src_p, dst_p, zeros)
  return _pool_head(p2[0], p2[1], xr2, batch3d, W3, b3)
